# Initial kernel scaffold; baseline (speedup 1.0000x reference)
import jax
import jax.numpy as jnp
from jax.experimental import pallas as pl


def kernel(positions, node_features, global_features, species_embed, W_rbf, W_upd, w_vec, w_glob, final_scaling):
    raise NotImplementedError("write your pallas kernel here")


# speedup vs baseline: 331.6690x; 331.6690x over previous
import jax
import jax.numpy as jnp
from jax.experimental import pallas as pl
from jax.experimental.pallas import tpu as pltpu

_N = 1024
_H = 16
_K = 8
_R_MAX = 5.0
_EPS = 1e-8
_AVG_NB = float(_N - 1)
_T = 2


def _mace_body(pos_ref, posT_ref, nf_ref, se_ref, wrbf_ref, wupd_ref,
               wvec_ref, gf_ref, wglob_ref, fs_ref, out_ref,
               h_scr, vec_scr):
    t = pl.program_id(0)

    @pl.when(t == 0)
    def _init():
        onehot = (jax.lax.broadcasted_iota(jnp.int32, (_N, 8), 1)
                  == nf_ref[:]).astype(jnp.float32)
        h_scr[:] = jnp.dot(onehot, se_ref[:],
                           preferred_element_type=jnp.float32)
        vec_scr[:] = jnp.zeros((_N, 3), jnp.float32)

    pos = pos_ref[:]
    posT = posT_ref[:]
    h = h_scr[:]

    d2 = jnp.full((_N, _N), _EPS, jnp.float32)
    for j in range(3):
        dj = pos[:, j:j + 1] - posT[j:j + 1, :]
        d2 = d2 + dj * dj
    length = jnp.sqrt(d2)

    cut = 0.5 * (jnp.cos(jnp.pi * jnp.clip(length * (1.0 / _R_MAX),
                                           0.0, 1.0)) + 1.0)
    rr = jax.lax.broadcasted_iota(jnp.int32, (_N, _N), 0)
    ss = jax.lax.broadcasted_iota(jnp.int32, (_N, _N), 1)
    cut = jnp.where(rr == ss, 0.0, cut)

    wrbf = wrbf_ref[0]
    wvec = wvec_ref[0]
    m_kc = wrbf * wvec
    qT = jax.lax.dot_general(m_kc, h, (((1,), (1,)), ((), ())),
                             preferred_element_type=jnp.float32)

    agg = jnp.zeros((_N, _H), jnp.float32)
    tacc = jnp.zeros((_N, _N), jnp.float32)
    for k in range(_K):
        mu = _R_MAX * k / (_K - 1)
        rbf = jnp.exp(-((length - mu) ** 2)) * cut
        agg = agg + jnp.dot(rbf, h,
                            preferred_element_type=jnp.float32) * wrbf[k, :]
        tacc = tacc + rbf * qT[k:k + 1, :]

    agg = agg * (1.0 / _AVG_NB)
    h_scr[:] = jnp.tanh(jnp.dot(agg, wupd_ref[0],
                                preferred_element_type=jnp.float32)) + h

    tmat = tacc / length
    rowsum = jnp.sum(tmat, axis=1, keepdims=True)
    tp = jnp.dot(tmat, pos, preferred_element_type=jnp.float32)
    vec_scr[:] = vec_scr[:] + (rowsum * pos - tp) * (1.0 / _AVG_NB)

    @pl.when(t == _T - 1)
    def _fin():
        gate = 1.0 + jnp.tanh(jnp.sum(gf_ref[:] * wglob_ref[:]))
        out_ref[:] = (vec_scr[:] * gate - pos) * fs_ref[0, 0]


def kernel(positions, node_features, global_features, species_embed,
           W_rbf, W_upd, w_vec, w_glob, final_scaling):
    pos = positions.astype(jnp.float32)
    posT = pos.T
    nf = node_features.astype(jnp.int32).reshape(_N, 1)
    se = jnp.zeros((8, _H), jnp.float32).at[:species_embed.shape[0]].set(
        species_embed.astype(jnp.float32))
    wrbf = W_rbf.astype(jnp.float32)
    wupd = W_upd.astype(jnp.float32)
    wvec = w_vec.astype(jnp.float32).reshape(_T, 1, _H)
    gf = global_features.astype(jnp.float32).reshape(1, -1)
    wglob = w_glob.astype(jnp.float32).reshape(1, -1)
    fs = final_scaling.astype(jnp.float32).reshape(1, 1)

    grid = (_T,)
    out = pl.pallas_call(
        _mace_body,
        grid=grid,
        in_specs=[
            pl.BlockSpec((_N, 3), lambda t: (0, 0)),
            pl.BlockSpec((3, _N), lambda t: (0, 0)),
            pl.BlockSpec((_N, 1), lambda t: (0, 0)),
            pl.BlockSpec((8, _H), lambda t: (0, 0)),
            pl.BlockSpec((1, _K, _H), lambda t: (t, 0, 0)),
            pl.BlockSpec((1, _H, _H), lambda t: (t, 0, 0)),
            pl.BlockSpec((1, 1, _H), lambda t: (t, 0, 0)),
            pl.BlockSpec((1, gf.shape[1]), lambda t: (0, 0)),
            pl.BlockSpec((1, wglob.shape[1]), lambda t: (0, 0)),
            pl.BlockSpec((1, 1), lambda t: (0, 0)),
        ],
        out_specs=pl.BlockSpec((_N, 3), lambda t: (0, 0)),
        out_shape=jax.ShapeDtypeStruct((_N, 3), jnp.float32),
        scratch_shapes=[
            pltpu.VMEM((_N, _H), jnp.float32),
            pltpu.VMEM((_N, 3), jnp.float32),
        ],
    )(pos, posT, nf, se, wrbf, wupd, wvec, gf, wglob, fs)
    return out
